# baseline (speedup 1.0000x reference)
import jax
import jax.numpy as jnp
from jax.experimental import pallas as pl


def _fused_norm_matmul_kernel(x_ref, t_ref, o_ref):
    x = x_ref[...].astype(jnp.bfloat16)
    acc = jnp.dot(x, t_ref[...], preferred_element_type=jnp.float32)
    n = o_ref.shape[1]
    count = acc[:, n : n + 1]
    count = jnp.where(count == 0.0, 1.0, count)
    o_ref[...] = acc[:, :n] / count


def kernel(concept_seq, table, domain):
    M, K = concept_seq.shape
    Kt, N = table.shape
    BM = 512
    taug = jnp.concatenate(
        [table, jnp.ones((Kt, 128), dtype=table.dtype)], axis=1
    ).astype(jnp.bfloat16)
    grid = (M // BM,)
    out = pl.pallas_call(
        _fused_norm_matmul_kernel,
        grid=grid,
        in_specs=[
            pl.BlockSpec((BM, K), lambda i: (i, 0)),
            pl.BlockSpec((Kt, N + 128), lambda i: (0, 0)),
        ],
        out_specs=pl.BlockSpec((BM, N), lambda i: (i, 0)),
        out_shape=jax.ShapeDtypeStruct((M, N), jnp.float32),
    )(concept_seq, taug)
    return out


# speedup vs baseline: 1.1686x; 1.1686x over previous
import jax
import jax.numpy as jnp
from jax.experimental import pallas as pl
from jax.experimental.pallas import tpu as pltpu


def _fused_norm_matmul_kernel(x_ref, t_ref, o_ref, taug_ref):
    n = o_ref.shape[1]

    @pl.when(pl.program_id(0) == 0)
    def _build_taug():
        taug_ref[:, :n] = t_ref[...].astype(jnp.bfloat16)
        taug_ref[:, n:] = jnp.ones(
            (t_ref.shape[0], 128), dtype=jnp.bfloat16
        )

    x = x_ref[...].astype(jnp.bfloat16)
    acc = jnp.dot(x, taug_ref[...], preferred_element_type=jnp.float32)
    count = acc[:, n : n + 1]
    count = jnp.where(count == 0.0, 1.0, count)
    o_ref[...] = acc[:, :n] / count


def kernel(concept_seq, table, domain):
    M, K = concept_seq.shape
    Kt, N = table.shape
    BM = 512
    grid = (M // BM,)
    out = pl.pallas_call(
        _fused_norm_matmul_kernel,
        grid=grid,
        in_specs=[
            pl.BlockSpec((BM, K), lambda i: (i, 0)),
            pl.BlockSpec((Kt, N), lambda i: (0, 0)),
        ],
        out_specs=pl.BlockSpec((BM, N), lambda i: (i, 0)),
        out_shape=jax.ShapeDtypeStruct((M, N), jnp.float32),
        scratch_shapes=[pltpu.VMEM((Kt, N + 128), jnp.bfloat16)],
    )(concept_seq, table)
    return out


# speedup vs baseline: 1.7016x; 1.4560x over previous
import jax
import jax.numpy as jnp
from jax.experimental import pallas as pl


def _fused_norm_matmul_kernel(x_ref, t_ref, o_ref):
    x = x_ref[...]
    count = jnp.sum(x, axis=1, keepdims=True)
    count = jnp.where(count == 0.0, 1.0, count)
    acc = jnp.dot(
        x.astype(jnp.bfloat16),
        t_ref[...].astype(jnp.bfloat16),
        preferred_element_type=jnp.float32,
    )
    o_ref[...] = acc / count


def kernel(concept_seq, table, domain):
    M, K = concept_seq.shape
    Kt, N = table.shape
    BM = 2048
    grid = (M // BM,)
    out = pl.pallas_call(
        _fused_norm_matmul_kernel,
        grid=grid,
        in_specs=[
            pl.BlockSpec((BM, K), lambda i: (i, 0)),
            pl.BlockSpec((Kt, N), lambda i: (0, 0)),
        ],
        out_specs=pl.BlockSpec((BM, N), lambda i: (i, 0)),
        out_shape=jax.ShapeDtypeStruct((M, N), jnp.float32),
    )(concept_seq, table)
    return out
